# baseline (speedup 1.0000x reference)
import functools

import jax
import jax.numpy as jnp
from jax import lax
from jax.experimental import pallas as pl
from jax.experimental.pallas import tpu as pltpu
from jax.experimental.pallas import tpu_sc as plsc

_NUM_USER = 1000000
_EMBED_DIM = 32
_BATCH = 16384

_CHUNK = 128


def _make_gather():
    info = plsc.get_sparse_core_info()
    nc, ns = info.num_cores, info.num_subcores
    nw = nc * ns
    b_per_w = _BATCH // nw
    n_chunks = b_per_w // _CHUNK
    mesh = plsc.VectorSubcoreMesh(core_axis_name="c", subcore_axis_name="s")

    @functools.partial(
        pl.kernel,
        mesh=mesh,
        compiler_params=pltpu.CompilerParams(use_tc_tiling_on_sc=False),
        out_type=jax.ShapeDtypeStruct((_BATCH, _EMBED_DIM), jnp.float32),
        scratch_types=[
            pltpu.VMEM((n_chunks, _CHUNK), jnp.int32),
            pltpu.VMEM((b_per_w, _EMBED_DIM), jnp.float32),
            pltpu.SemaphoreType.DMA,
        ],
    )
    def k(idx_hbm, table_hbm, out_hbm, idx_v, rows_v, sem):
        wid = lax.axis_index("s") * nc + lax.axis_index("c")
        base = wid * b_per_w
        pltpu.sync_copy(idx_hbm.at[wid], idx_v)
        copies = []
        for j in range(n_chunks):
            copies.append(
                pltpu.async_copy(
                    table_hbm.at[idx_v.at[j]],
                    rows_v.at[pl.ds(j * _CHUNK, _CHUNK)],
                    sem,
                )
            )
        for cp in copies:
            cp.wait()
        pltpu.sync_copy(rows_v, out_hbm.at[pl.ds(base, b_per_w)])

    return k, nw, n_chunks


_gather, _NW, _NCHUNKS = _make_gather()


def kernel(user_idx, table):
    idx3 = user_idx.astype(jnp.int32).reshape(_NW, _NCHUNKS, _CHUNK)
    return _gather(idx3, table)


# speedup vs baseline: 4.0892x; 4.0892x over previous
import functools

import jax
import jax.numpy as jnp
from jax import lax
from jax.experimental import pallas as pl
from jax.experimental.pallas import tpu as pltpu
from jax.experimental.pallas import tpu_sc as plsc

_NUM_USER = 1000000
_EMBED_DIM = 32
_BATCH = 16384
_LANES = 16
_RING = 16


def _make_gather():
    info = plsc.get_sparse_core_info()
    nc, ns = info.num_cores, info.num_subcores
    nw = nc * ns
    b_per_w = _BATCH // nw
    n_groups = b_per_w // _RING
    mesh = plsc.VectorSubcoreMesh(core_axis_name="c", subcore_axis_name="s")

    @functools.partial(
        pl.kernel,
        mesh=mesh,
        compiler_params=pltpu.CompilerParams(needs_layout_passes=False),
        out_type=jax.ShapeDtypeStruct((_EMBED_DIM, _BATCH), jnp.float32),
        scratch_types=[
            pltpu.VMEM((b_per_w,), jnp.int32),
            pltpu.VMEM((_EMBED_DIM, b_per_w), jnp.float32),
            [pltpu.VMEM((_EMBED_DIM, 128), jnp.float32) for _ in range(_RING)],
            [pltpu.SemaphoreType.DMA for _ in range(_RING)],
        ],
    )
    def k(idx_hbm, tt_hbm, out_hbm, idx_s, out_v, slabs, sems):
        wid = lax.axis_index("s") * nc + lax.axis_index("c")
        base = wid * b_per_w
        pltpu.sync_copy(idx_hbm.at[pl.ds(base, b_per_w)], idx_s)

        d_lo = lax.iota(jnp.int32, _LANES)
        d_hi = d_lo + _LANES

        def fetch(u, r):
            c = pl.multiple_of((u >> 7) << 7, 128)
            pltpu.async_copy(tt_hbm.at[:, pl.ds(c, 128)], slabs[r], sems[r])

        def extract(u, i, r):
            l_vec = jnp.full((_LANES,), u & 127, jnp.int32)
            i_vec = jnp.full((_LANES,), i, jnp.int32)
            lo = plsc.load_gather(slabs[r], [d_lo, l_vec])
            hi = plsc.load_gather(slabs[r], [d_hi, l_vec])
            plsc.store_scatter(out_v, [d_lo, i_vec], lo)
            plsc.store_scatter(out_v, [d_hi, i_vec], hi)

        idx0 = idx_s[pl.ds(0, _RING)]
        for r in range(_RING):
            fetch(idx0[r], r)

        def round_body(j, _):
            idx_cur = idx_s[pl.ds(j * _RING, _RING)]
            j_nxt = lax.rem(j + 1, n_groups)
            idx_nxt = idx_s[pl.ds(j_nxt * _RING, _RING)]
            for r in range(_RING):
                pltpu.make_async_copy(
                    tt_hbm.at[:, pl.ds(0, 128)], slabs[r], sems[r]
                ).wait()
                extract(idx_cur[r], j * _RING + r, r)

                @pl.when(j < n_groups - 1)
                def _():
                    fetch(idx_nxt[r], r)

            return _

        lax.fori_loop(0, n_groups, round_body, None)
        pltpu.sync_copy(out_v, out_hbm.at[:, pl.ds(base, b_per_w)])

    return k


_gather = _make_gather()


def kernel(user_idx, table):
    out_t = _gather(user_idx.astype(jnp.int32), table.T)
    return out_t.T


# speedup vs baseline: 4.0973x; 1.0020x over previous
import functools

import jax
import jax.numpy as jnp
from jax import lax
from jax.experimental import pallas as pl
from jax.experimental.pallas import tpu as pltpu
from jax.experimental.pallas import tpu_sc as plsc

_NUM_USER = 1000000
_EMBED_DIM = 32
_BATCH = 16384
_LANES = 16
_RING = 16


def _make_gather():
    info = plsc.get_sparse_core_info()
    nc, ns = info.num_cores, info.num_subcores
    nw = nc * ns
    b_per_w = _BATCH // nw
    n_groups = b_per_w // _RING
    mesh = plsc.VectorSubcoreMesh(core_axis_name="c", subcore_axis_name="s")

    @functools.partial(
        pl.kernel,
        mesh=mesh,
        compiler_params=pltpu.CompilerParams(needs_layout_passes=False),
        out_type=jax.ShapeDtypeStruct((_EMBED_DIM, _BATCH), jnp.float32),
        scratch_types=[
            pltpu.VMEM((b_per_w,), jnp.int32),
            pltpu.VMEM((_EMBED_DIM, b_per_w), jnp.float32),
            [pltpu.VMEM((_EMBED_DIM, 128), jnp.float32) for _ in range(_RING)],
            [pltpu.SemaphoreType.DMA for _ in range(_RING)],
        ],
    )
    def k(idx_hbm, tt_hbm, out_hbm, idx_s, out_v, slabs, sems):
        wid = lax.axis_index("s") * nc + lax.axis_index("c")
        base = wid * b_per_w
        pltpu.sync_copy(idx_hbm.at[pl.ds(base, b_per_w)], idx_s)

        d_lo = lax.iota(jnp.int32, _LANES)
        d_hi = d_lo + _LANES

        def fetch(u, r):
            c = pl.multiple_of((u >> 7) << 7, 128)
            for g in range(4):
                pltpu.async_copy(
                    tt_hbm.at[pl.ds(8 * g, 8), pl.ds(c, 128)],
                    slabs[r].at[pl.ds(8 * g, 8), :],
                    sems[r],
                )

        def extract(u, i, r):
            l_vec = jnp.full((_LANES,), u & 127, jnp.int32)
            i_vec = jnp.full((_LANES,), i, jnp.int32)
            lo = plsc.load_gather(slabs[r], [d_lo, l_vec])
            hi = plsc.load_gather(slabs[r], [d_hi, l_vec])
            plsc.store_scatter(out_v, [d_lo, i_vec], lo)
            plsc.store_scatter(out_v, [d_hi, i_vec], hi)

        idx0 = idx_s[pl.ds(0, _RING)]
        for r in range(_RING):
            fetch(idx0[r], r)

        def round_body(j, _):
            idx_cur = idx_s[pl.ds(j * _RING, _RING)]
            j_nxt = lax.rem(j + 1, n_groups)
            idx_nxt = idx_s[pl.ds(j_nxt * _RING, _RING)]
            for r in range(_RING):
                for g in range(4):
                    pltpu.make_async_copy(
                        tt_hbm.at[pl.ds(0, 8), pl.ds(0, 128)],
                        slabs[r].at[pl.ds(0, 8), :],
                        sems[r],
                    ).wait()
                extract(idx_cur[r], j * _RING + r, r)

                @pl.when(j < n_groups - 1)
                def _():
                    fetch(idx_nxt[r], r)

            return _

        lax.fori_loop(0, n_groups, round_body, None)
        pltpu.sync_copy(out_v, out_hbm.at[:, pl.ds(base, b_per_w)])

    return k


_gather = _make_gather()


def kernel(user_idx, table):
    out_t = _gather(user_idx.astype(jnp.int32), table.T)
    return out_t.T
